# baseline (speedup 1.0000x reference)
import functools

import jax
import jax.numpy as jnp
from jax import lax
from jax.experimental import pallas as pl
from jax.experimental.pallas import tpu as pltpu
from jax.experimental.pallas import tpu_sc as plsc

VOCAB = 128
DIM = 128
BATCH = 16384


@functools.cache
def _build():
    info = plsc.get_sparse_core_info()
    nc, ns = info.num_cores, info.num_subcores
    nw = nc * ns
    b_per_w = BATCH // nw
    chunk = 32
    nchunk = b_per_w // chunk

    mesh = plsc.VectorSubcoreMesh(core_axis_name="c", subcore_axis_name="s")

    @functools.partial(
        pl.kernel,
        mesh=mesh,
        out_type=jax.ShapeDtypeStruct((BATCH, DIM), jnp.float32),
        scratch_types=[
            pltpu.VMEM((b_per_w,), jnp.int32),
            pltpu.VMEM((chunk, DIM), jnp.float32),
            pltpu.VMEM((chunk, DIM), jnp.float32),
            pltpu.VMEM_SHARED((VOCAB, DIM), jnp.float32),
            pltpu.SemaphoreType.DMA,
            pltpu.SemaphoreType.DMA,
            pltpu.SemaphoreType.DMA,
            pltpu.SemaphoreType.DMA,
            pltpu.SemaphoreType.DMA,
            pltpu.SemaphoreType.DMA,
        ],
    )
    def gather_kernel(x_hbm, table_hbm, out_hbm, idx_v, rows0, rows1,
                      table_sh, g0, g1, w0, w1, isem, tsem):
        s = lax.axis_index("s")
        wid = s * nc + lax.axis_index("c")
        base = wid * b_per_w
        bufs = (rows0, rows1)
        gsems = (g0, g1)
        wsems = (w0, w1)
        tcopy = pltpu.make_async_copy(table_hbm, table_sh, tsem)

        @pl.when(s == 0)
        def _():
            tcopy.start()

        icopy = pltpu.make_async_copy(
            x_hbm.at[pl.ds(base + chunk, b_per_w - chunk)],
            idx_v.at[pl.ds(chunk, b_per_w - chunk)], isem)
        icopy.start()
        pltpu.sync_copy(x_hbm.at[pl.ds(base, chunk)], idx_v.at[pl.ds(0, chunk)])

        @pl.when(s == 0)
        def _():
            tcopy.wait()

        plsc.subcore_barrier()

        gcs = []
        wcs = []
        for k in range(nchunk):
            b = k % 2
            if k == 1:
                icopy.wait()
            if k >= 2:
                wcs[k - 2].wait()
            gc = pltpu.make_async_copy(
                table_sh.at[idx_v.at[pl.ds(k * chunk, chunk)]], bufs[b], gsems[b])
            gc.start()
            gcs.append(gc)
            if k >= 1:
                gcs[k - 1].wait()
                wc = pltpu.make_async_copy(
                    bufs[(k - 1) % 2], out_hbm.at[pl.ds(base + (k - 1) * chunk, chunk)],
                    wsems[(k - 1) % 2])
                wc.start()
                wcs.append(wc)
        gcs[-1].wait()
        wc = pltpu.make_async_copy(
            bufs[(nchunk - 1) % 2],
            out_hbm.at[pl.ds(base + (nchunk - 1) * chunk, chunk)],
            wsems[(nchunk - 1) % 2])
        wc.start()
        wcs.append(wc)
        wcs[-2].wait()
        wcs[-1].wait()

    return gather_kernel


def kernel(x, table):
    return _build()(x.astype(jnp.int32), table)


# speedup vs baseline: 1.0216x; 1.0216x over previous
import functools

import jax
import jax.numpy as jnp
from jax import lax
from jax.experimental import pallas as pl
from jax.experimental.pallas import tpu as pltpu
from jax.experimental.pallas import tpu_sc as plsc

VOCAB = 128
DIM = 128
BATCH = 16384


@functools.cache
def _build():
    info = plsc.get_sparse_core_info()
    nc, ns = info.num_cores, info.num_subcores
    nw = nc * ns
    b_per_w = BATCH // nw
    chunk = 64
    nchunk = b_per_w // chunk
    nbuf = 4

    mesh = plsc.VectorSubcoreMesh(core_axis_name="c", subcore_axis_name="s")

    @functools.partial(
        pl.kernel,
        mesh=mesh,
        out_type=jax.ShapeDtypeStruct((BATCH, DIM), jnp.float32),
        scratch_types=(
            [pltpu.VMEM((b_per_w,), jnp.int32)]
            + [pltpu.VMEM((chunk, DIM), jnp.float32) for _ in range(nbuf)]
            + [pltpu.VMEM_SHARED((VOCAB, DIM), jnp.float32)]
            + [pltpu.SemaphoreType.DMA for _ in range(2 * nbuf + 2)]
        ),
    )
    def gather_kernel(x_hbm, table_hbm, out_hbm, idx_v, *rest):
        bufs = rest[:nbuf]
        table_sh = rest[nbuf]
        gsems = rest[nbuf + 1:2 * nbuf + 1]
        wsems = rest[2 * nbuf + 1:3 * nbuf + 1]
        isem = rest[3 * nbuf + 1]
        tsem = rest[3 * nbuf + 2]
        s = lax.axis_index("s")
        wid = s * nc + lax.axis_index("c")
        base = wid * b_per_w
        tcopy = pltpu.make_async_copy(table_hbm, table_sh, tsem)

        @pl.when(s == 0)
        def _():
            tcopy.start()

        icopy = pltpu.make_async_copy(
            x_hbm.at[pl.ds(base + chunk, b_per_w - chunk)],
            idx_v.at[pl.ds(chunk, b_per_w - chunk)], isem)
        icopy.start()
        pltpu.sync_copy(x_hbm.at[pl.ds(base, chunk)], idx_v.at[pl.ds(0, chunk)])

        @pl.when(s == 0)
        def _():
            tcopy.wait()

        plsc.subcore_barrier()

        gcs = []
        wcs = []
        for k in range(nchunk):
            if k == 1:
                icopy.wait()
            if k >= 1:
                gcs[k - 1].wait()
                wc = pltpu.make_async_copy(
                    bufs[(k - 1) % nbuf],
                    out_hbm.at[pl.ds(base + (k - 1) * chunk, chunk)],
                    wsems[(k - 1) % nbuf])
                wc.start()
                wcs.append(wc)
            if k >= nbuf:
                wcs[k - nbuf].wait()
            gc = pltpu.make_async_copy(
                table_sh.at[idx_v.at[pl.ds(k * chunk, chunk)]],
                bufs[k % nbuf], gsems[k % nbuf])
            gc.start()
            gcs.append(gc)
        gcs[-1].wait()
        wc = pltpu.make_async_copy(
            bufs[(nchunk - 1) % nbuf],
            out_hbm.at[pl.ds(base + (nchunk - 1) * chunk, chunk)],
            wsems[(nchunk - 1) % nbuf])
        wc.start()
        wcs.append(wc)
        for j in range(nbuf - 1):
            wcs[nchunk - nbuf + j].wait()
        wcs[-1].wait()

    return gather_kernel


def kernel(x, table):
    return _build()(x.astype(jnp.int32), table)


# speedup vs baseline: 1.0269x; 1.0051x over previous
import functools

import jax
import jax.numpy as jnp
from jax import lax
from jax.experimental import pallas as pl
from jax.experimental.pallas import tpu as pltpu
from jax.experimental.pallas import tpu_sc as plsc

VOCAB = 128
DIM = 128
BATCH = 16384


@functools.cache
def _build():
    info = plsc.get_sparse_core_info()
    nc, ns = info.num_cores, info.num_subcores
    nw = nc * ns
    b_per_w = BATCH // nw
    chunk = 128
    nchunk = b_per_w // chunk

    mesh = plsc.VectorSubcoreMesh(core_axis_name="c", subcore_axis_name="s")

    @functools.partial(
        pl.kernel,
        mesh=mesh,
        out_type=jax.ShapeDtypeStruct((BATCH, DIM), jnp.float32),
        scratch_types=[
            pltpu.VMEM((b_per_w,), jnp.int32),
            pltpu.VMEM((chunk, DIM), jnp.float32),
            pltpu.VMEM((chunk, DIM), jnp.float32),
            pltpu.VMEM_SHARED((VOCAB, DIM), jnp.float32),
            pltpu.SemaphoreType.DMA,
            pltpu.SemaphoreType.DMA,
            pltpu.SemaphoreType.DMA,
            pltpu.SemaphoreType.DMA,
            pltpu.SemaphoreType.DMA,
            pltpu.SemaphoreType.DMA,
        ],
    )
    def gather_kernel(x_hbm, table_hbm, out_hbm, idx_v, rows0, rows1,
                      table_sh, g0, g1, w0, w1, isem, tsem):
        s = lax.axis_index("s")
        wid = s * nc + lax.axis_index("c")
        base = wid * b_per_w
        bufs = (rows0, rows1)
        gsems = (g0, g1)
        wsems = (w0, w1)
        tcopy = pltpu.make_async_copy(table_hbm, table_sh, tsem)

        @pl.when(s == 0)
        def _():
            tcopy.start()

        icopy = pltpu.make_async_copy(
            x_hbm.at[pl.ds(base + chunk, b_per_w - chunk)],
            idx_v.at[pl.ds(chunk, b_per_w - chunk)], isem)
        icopy.start()
        pltpu.sync_copy(x_hbm.at[pl.ds(base, chunk)], idx_v.at[pl.ds(0, chunk)])

        @pl.when(s == 0)
        def _():
            tcopy.wait()

        plsc.subcore_barrier()

        gcs = []
        wcs = []
        for k in range(nchunk):
            b = k % 2
            if k == 1:
                icopy.wait()
            if k >= 2:
                wcs[k - 2].wait()
            gc = pltpu.make_async_copy(
                table_sh.at[idx_v.at[pl.ds(k * chunk, chunk)]], bufs[b], gsems[b])
            gc.start()
            gcs.append(gc)
            if k >= 1:
                gcs[k - 1].wait()
                wc = pltpu.make_async_copy(
                    bufs[(k - 1) % 2], out_hbm.at[pl.ds(base + (k - 1) * chunk, chunk)],
                    wsems[(k - 1) % 2])
                wc.start()
                wcs.append(wc)
        gcs[-1].wait()
        wc = pltpu.make_async_copy(
            bufs[(nchunk - 1) % 2],
            out_hbm.at[pl.ds(base + (nchunk - 1) * chunk, chunk)],
            wsems[(nchunk - 1) % 2])
        wc.start()
        wcs.append(wc)
        wcs[-2].wait()
        wcs[-1].wait()

    return gather_kernel


def kernel(x, table):
    return _build()(x.astype(jnp.int32), table)


# speedup vs baseline: 1.0285x; 1.0015x over previous
import functools

import jax
import jax.numpy as jnp
from jax import lax
from jax.experimental import pallas as pl
from jax.experimental.pallas import tpu as pltpu
from jax.experimental.pallas import tpu_sc as plsc

VOCAB = 128
DIM = 128
BATCH = 16384


@functools.cache
def _build():
    info = plsc.get_sparse_core_info()
    nc, ns = info.num_cores, info.num_subcores
    nw = nc * ns
    b_per_w = BATCH // nw
    chunk = 64
    nchunk = b_per_w // chunk

    mesh = plsc.VectorSubcoreMesh(core_axis_name="c", subcore_axis_name="s")

    @functools.partial(
        pl.kernel,
        mesh=mesh,
        out_type=jax.ShapeDtypeStruct((BATCH, DIM), jnp.float32),
        scratch_types=[
            pltpu.VMEM((b_per_w,), jnp.int32),
            pltpu.VMEM((chunk, DIM), jnp.float32),
            pltpu.VMEM((chunk, DIM), jnp.float32),
            pltpu.VMEM_SHARED((VOCAB, DIM), jnp.float32),
            pltpu.SemaphoreType.DMA,
            pltpu.SemaphoreType.DMA,
            pltpu.SemaphoreType.DMA,
            pltpu.SemaphoreType.DMA,
            pltpu.SemaphoreType.DMA,
            pltpu.SemaphoreType.DMA,
        ],
    )
    def gather_kernel(x_hbm, table_hbm, out_hbm, idx_v, rows0, rows1,
                      table_sh, g0, g1, w0, w1, isem, tsem):
        s = lax.axis_index("s")
        wid = s * nc + lax.axis_index("c")
        base = wid * b_per_w
        bufs = (rows0, rows1)
        gsems = (g0, g1)
        wsems = (w0, w1)
        tcopy = pltpu.make_async_copy(table_hbm, table_sh, tsem)

        @pl.when(s == 0)
        def _():
            tcopy.start()

        icopy = pltpu.make_async_copy(
            x_hbm.at[pl.ds(base + chunk, b_per_w - chunk)],
            idx_v.at[pl.ds(chunk, b_per_w - chunk)], isem)
        icopy.start()
        pltpu.sync_copy(x_hbm.at[pl.ds(base, chunk)], idx_v.at[pl.ds(0, chunk)])

        @pl.when(s == 0)
        def _():
            tcopy.wait()

        plsc.subcore_barrier()

        gcs = []
        wcs = []
        for k in range(nchunk):
            b = k % 2
            if k == 1:
                icopy.wait()
            if k >= 2:
                wcs[k - 2].wait()
            gc = pltpu.make_async_copy(
                table_sh.at[idx_v.at[pl.ds(k * chunk, chunk)]], bufs[b], gsems[b])
            gc.start()
            gcs.append(gc)
            if k >= 1:
                gcs[k - 1].wait()
                wc = pltpu.make_async_copy(
                    bufs[(k - 1) % 2], out_hbm.at[pl.ds(base + (k - 1) * chunk, chunk)],
                    wsems[(k - 1) % 2])
                wc.start()
                wcs.append(wc)
        gcs[-1].wait()
        wc = pltpu.make_async_copy(
            bufs[(nchunk - 1) % 2],
            out_hbm.at[pl.ds(base + (nchunk - 1) * chunk, chunk)],
            wsems[(nchunk - 1) % 2])
        wc.start()
        wcs.append(wc)
        wcs[-2].wait()
        wcs[-1].wait()

    return gather_kernel


def kernel(x, table):
    return _build()(x.astype(jnp.int32), table)
